# baseline (speedup 1.0000x reference)
import functools

import jax
import jax.numpy as jnp
from jax.experimental import pallas as pl
from jax.experimental.pallas import tpu as pltpu

N_BINS = 15
_BIN_PAD = 16
_LANES = 128
_NQ = 4
_WIDE = _NQ * N_BINS * _LANES


def _ece_kernel(logits_ref, labels_ref, out_ref,
                acc_ref, maxlab_ref, *, n_total):
    step = pl.program_id(0)
    nsteps = pl.num_programs(0)

    @pl.when(step == 0)
    def _init():
        acc_ref[...] = jnp.zeros((8, _WIDE), jnp.float32)
        maxlab_ref[0] = 0

    x = logits_ref[...]
    bn, c = x.shape
    m = jnp.max(x, axis=1, keepdims=True)
    e = jnp.exp(x - m)
    s = jnp.sum(e, axis=1, keepdims=True)
    conf = e * (1.0 / s)

    idx = jnp.ceil(conf * jnp.float32(N_BINS)) - 1.0
    idx = jnp.where(conf > 0.0, idx, -1.0)

    pad_cfg = ((0, 0, 0), (0, _LANES - c, 0))
    idx_p = jax.lax.pad(idx, jnp.float32(-1.0), pad_cfg)
    conf_p = jax.lax.pad(conf, jnp.float32(0.0), pad_cfg)

    lbl = labels_ref[...]
    maxlab_ref[0] = jnp.maximum(maxlab_ref[0], jnp.max(lbl))
    cls_iota = jax.lax.broadcasted_iota(jnp.int32, (bn, _LANES), 1)
    onehot_bf = (lbl == cls_iota).astype(jnp.bfloat16)

    conf_hi = conf_p.astype(jnp.bfloat16)
    conf_lo = (conf_p - conf_hi.astype(jnp.float32)).astype(jnp.bfloat16)

    idx_bf = idx_p.astype(jnp.bfloat16)
    zero_bf = jnp.bfloat16(0.0)
    one_bf = jnp.bfloat16(1.0)
    cnt_chunks, corr_chunks, hi_chunks, lo_chunks = [], [], [], []
    for b in range(N_BINS):
        eq = idx_bf == jnp.bfloat16(b)
        cnt_chunks.append(jnp.where(eq, one_bf, zero_bf))
        corr_chunks.append(jnp.where(eq, onehot_bf, zero_bf))
        hi_chunks.append(jnp.where(eq, conf_hi, zero_bf))
        lo_chunks.append(jnp.where(eq, conf_lo, zero_bf))
    wide = jnp.concatenate(
        cnt_chunks + corr_chunks + hi_chunks + lo_chunks, axis=1)
    ones = jnp.ones((1, bn), jnp.bfloat16)
    row = jax.lax.dot_general(
        ones, wide, dimension_numbers=(((1,), (0,)), ((), ())),
        preferred_element_type=jnp.float32)
    acc_ref[0:1, :] += row

    @pl.when(step == nsteps - 1)
    def _finalize():
        w = N_BINS * _LANES
        count = acc_ref[0:1, 0:w].reshape(N_BINS, _LANES)
        corr = acc_ref[0:1, w:2 * w].reshape(N_BINS, _LANES)
        confsum = (acc_ref[0:1, 2 * w:3 * w].reshape(N_BINS, _LANES)
                   + acc_ref[0:1, 3 * w:4 * w].reshape(N_BINS, _LANES))
        num_classes = (maxlab_ref[0] + 1).astype(jnp.float32)
        prop = count * jnp.float32(1.0 / n_total)
        safe = jnp.maximum(count, 1.0)
        acc_in_bin = corr / safe
        avg_conf = confsum / safe
        term = jnp.where(count > 0.0,
                         jnp.abs(avg_conf - acc_in_bin) * prop, 0.0)
        class_sce = jnp.sum(term, axis=0, keepdims=True)
        cls = jax.lax.broadcasted_iota(jnp.int32, (1, _LANES), 1)
        mask = (cls < (maxlab_ref[0] + 1)).astype(jnp.float32)
        out_ref[...] = jnp.sum(class_sce * mask, keepdims=True) / num_classes


def kernel(logits, labels):
    n, c = logits.shape
    bn = n
    for cand in (2000, 1250, 1000, 625, 500, 400, 250, 200, 125, 100):
        if n % cand == 0 and cand % 8 == 0:
            bn = cand
            break
    grid = n // bn
    out = pl.pallas_call(
        functools.partial(_ece_kernel, n_total=n),
        grid=(grid,),
        in_specs=[
            pl.BlockSpec((bn, c), lambda i: (i, 0)),
            pl.BlockSpec((bn, 1), lambda i: (i, 0)),
        ],
        out_specs=pl.BlockSpec((1, 1), lambda i: (0, 0)),
        out_shape=jax.ShapeDtypeStruct((1, 1), jnp.float32),
        scratch_shapes=[
            pltpu.VMEM((8, _WIDE), jnp.float32),
            pltpu.SMEM((1,), jnp.int32),
        ],
        compiler_params=pltpu.CompilerParams(
            dimension_semantics=("arbitrary",)),
    )(logits, labels.reshape(n, 1))
    return out.reshape(())
